# baseline (speedup 1.0000x reference)
import functools

import jax
import jax.numpy as jnp
from jax import lax
from jax.experimental import pallas as pl
from jax.experimental.pallas import tpu as pltpu
from jax.experimental.pallas import tpu_sc as plsc

_NUM_HYPO = 5
_K = _NUM_HYPO - 1
_DIST_THRESH = 2.0
_TB = 512


def _topk_body(a_ref, c_ref, ids_ref, mask_ref):
    b = pl.program_id(0)
    n = c_ref.shape[2]
    cw = 128
    nch = n // cw
    ax = a_ref[0, :, 0:1]
    ay = a_ref[0, :, 1:2]
    cx = c_ref[0, 0:1, :]
    cy = c_ref[0, 1:2, :]
    dx = ax - cx
    dy = ay - cy
    dist = jnp.sqrt(dx * dx + dy * dy)
    col = lax.broadcasted_iota(jnp.int32, dist.shape, 1)
    lane = lax.broadcasted_iota(jnp.int32, (dist.shape[0], cw), 1).astype(
        jnp.float32
    )
    base = b * (n + 1)
    big = jnp.float32(2.0 * n)
    id_cols = []
    mask_cols = []
    for h in range(_K):
        best_v = dist[:, 0:cw]
        best_c = jnp.zeros((dist.shape[0], cw), jnp.float32)
        for j in range(1, nch):
            vj = dist[:, j * cw : (j + 1) * cw]
            pred = vj < best_v
            best_v = jnp.minimum(best_v, vj)
            best_c = jnp.where(pred, jnp.float32(j), best_c)
        m = jnp.min(best_v, axis=1, keepdims=True)
        gcol = best_c * cw + lane
        idxf = jnp.min(
            jnp.where(best_v == m, gcol, big), axis=1, keepdims=True
        )
        idx = idxf.astype(jnp.int32)
        valid = m < _DIST_THRESH
        id_cols.append(base + jnp.where(valid, idx, n))
        mask_cols.append(valid.astype(jnp.int32))
        if h < _K - 1:
            dist = jnp.where(col == idx, jnp.float32(jnp.inf), dist)
    ids_ref[0] = jnp.concatenate(id_cols, axis=1)
    mask_ref[0] = jnp.concatenate(mask_cols, axis=1)


def _topk_call(axy, cxy_t, interpret=False):
    B, T, _ = axy.shape
    N = cxy_t.shape[2]
    grid = (B, T // _TB)
    return pl.pallas_call(
        _topk_body,
        grid=grid,
        in_specs=[
            pl.BlockSpec((1, _TB, 2), lambda b, t: (b, t, 0)),
            pl.BlockSpec((1, 2, N), lambda b, t: (b, 0, 0)),
        ],
        out_specs=[
            pl.BlockSpec((1, _TB, _K), lambda b, t: (b, t, 0)),
            pl.BlockSpec((1, _TB, _K), lambda b, t: (b, t, 0)),
        ],
        out_shape=[
            jax.ShapeDtypeStruct((B, T, _K), jnp.int32),
            jax.ShapeDtypeStruct((B, T, _K), jnp.int32),
        ],
        interpret=interpret,
    )(axy, cxy_t)


def _make_sc_assemble(B, T, N, L):
    info = plsc.get_sparse_core_info()
    nc, ns = info.num_cores, info.num_subcores
    nw = nc * ns
    rpw = (B * T * _K) // nw
    ch = rpw // 128
    tq = T // 4
    rec = _NUM_HYPO * 8
    units2 = B * L
    mesh = plsc.VectorSubcoreMesh(core_axis_name="c", subcore_axis_name="s")

    @functools.partial(
        pl.kernel,
        mesh=mesh,
        out_type=[
            jax.ShapeDtypeStruct((B * (L + 1) * T * rec,), jnp.float32),
            jax.ShapeDtypeStruct((B * T * rec,), jnp.float32),
        ],
        scratch_types=[
            pltpu.VMEM((ch, 128), jnp.int32),
            pltpu.VMEM((rpw, 16), jnp.float32),
            pltpu.VMEM((T * 8,), jnp.float32),
            pltpu.VMEM((T * rec,), jnp.float32),
            pltpu.SemaphoreType.DMA,
        ],
        compiler_params=pltpu.CompilerParams(
            use_tc_tiling_on_sc=False, needs_layout_passes=False
        ),
    )
    def assemble_kernel(
        table_hbm, idx_hbm, td_hbm, traj_hbm, hyp_hbm, cand_hbm,
        idx_v, rows_v, tin_v, buf, sem,
    ):
        wid = lax.axis_index("s") * nc + lax.axis_index("c")
        lanes = lax.iota(jnp.int32, 16)
        lo8 = lanes < 8
        pat2 = jnp.where(lo8, lanes, lanes - 8 + rec)

        b1 = wid // 4
        t0 = (wid % 4) * tq
        pltpu.sync_copy(idx_hbm.at[wid], idx_v)
        copies = [
            pltpu.async_copy(
                table_hbm.at[idx_v.at[j]], rows_v.at[pl.ds(j * 128, 128)], sem
            )
            for j in range(ch)
        ]
        for c in copies:
            c.wait()
        pltpu.sync_copy(
            td_hbm.at[b1, pl.ds(t0 * 8, tq * 8)], tin_v.at[pl.ds(0, tq * 8)]
        )

        def p1_td(i, _):
            x = tin_v[pl.ds(i * 16, 16)]
            plsc.store_scatter(buf, [pat2 + i * (2 * rec)], x)
            return 0

        lax.fori_loop(0, tq // 2, p1_td, 0)

        def p1_box(t, _):
            for h in range(_K):
                x = rows_v[4 * t + h]
                plsc.store_scatter(
                    buf, [lanes + (t * rec + 8 * h + 8)], x, mask=lo8
                )
            return 0

        lax.fori_loop(0, tq, p1_box, 0)
        pltpu.sync_copy(
            buf.at[pl.ds(0, tq * rec)],
            cand_hbm.at[pl.ds(b1 * T * rec + t0 * rec, tq * rec)],
        )
        pltpu.sync_copy(
            buf.at[pl.ds(0, tq * rec)],
            hyp_hbm.at[pl.ds(b1 * (L + 1) * T * rec + t0 * rec, tq * rec)],
        )

        for k in range(-(-units2 // nw)):
            u = wid + k * nw

            @pl.when(u < units2)
            def _():
                b2 = u // L
                l2 = u % L
                pltpu.sync_copy(traj_hbm.at[b2, l2], tin_v)

                def p2(i, _):
                    x = tin_v[pl.ds(i * 16, 16)]
                    base = pat2 + i * (2 * rec)
                    for h in range(_NUM_HYPO):
                        plsc.store_scatter(buf, [base + 8 * h], x)
                    return 0

                lax.fori_loop(0, T // 2, p2, 0)
                pltpu.sync_copy(
                    buf,
                    hyp_hbm.at[
                        pl.ds((b2 * (L + 1) + l2 + 1) * T * rec, T * rec)
                    ],
                )

    return assemble_kernel


def kernel(transfered_det, det_boxes3d, traj):
    B, T, _ = transfered_det.shape
    N = det_boxes3d.shape[1]
    L = traj.shape[1]

    axy = transfered_det[:, :, :2]
    cxy_t = jnp.transpose(det_boxes3d[:, :, :2], (0, 2, 1))
    flat_ids, maskv = _topk_call(axy, cxy_t)

    table = jnp.pad(det_boxes3d, ((0, 0), (0, 1), (0, 9)))
    table = table.reshape(B * (N + 1), 16)

    nw = 32
    idx3 = flat_ids.reshape(nw, (B * T * _K) // (nw * 128), 128)
    hyp_flat, cand_flat = _make_sc_assemble(B, T, N, L)(
        table,
        idx3,
        transfered_det.reshape(B, T * 8),
        traj.reshape(B, L, T * 8),
    )
    hypotheses = hyp_flat.reshape(B, L + 1, T, _NUM_HYPO, 8)
    global_candidates = cand_flat.reshape(B, 1, T, _NUM_HYPO, 8)
    valid_mask = maskv != 0
    return (hypotheses, global_candidates, valid_mask)


# speedup vs baseline: 2.0051x; 2.0051x over previous
import functools

import jax
import jax.numpy as jnp
from jax import lax
from jax.experimental import pallas as pl
from jax.experimental.pallas import tpu as pltpu
from jax.experimental.pallas import tpu_sc as plsc

_NUM_HYPO = 5
_K = _NUM_HYPO - 1
_DIST_THRESH = 2.0
_TB = 512


def _topk_body(a_ref, c_ref, ids_ref, mask_ref):
    b = pl.program_id(0)
    n = c_ref.shape[1]
    cr = 128
    nch = n // cr
    tb = a_ref.shape[2]
    ax = a_ref[0, 0:1, :]
    ay = a_ref[0, 1:2, :]
    cx = c_ref[0, :, 0:1]
    cy = c_ref[0, :, 1:2]
    dx = cx - ax
    dy = cy - ay
    dist = jnp.sqrt(dx * dx + dy * dy)
    row = lax.broadcasted_iota(jnp.int32, dist.shape, 0)
    sub = lax.broadcasted_iota(jnp.int32, (cr, tb), 0).astype(jnp.float32)
    base = b * (n + 1)
    big = jnp.float32(2.0 * n)
    for h in range(_K):
        best_v = dist[0:cr, :]
        best_c = jnp.zeros((cr, tb), jnp.float32)
        for j in range(1, nch):
            vj = dist[j * cr : (j + 1) * cr, :]
            pred = vj < best_v
            best_v = jnp.minimum(best_v, vj)
            best_c = jnp.where(pred, jnp.float32(j), best_c)
        m = jnp.min(best_v, axis=0, keepdims=True)
        grow = best_c * cr + sub
        idxf = jnp.min(
            jnp.where(best_v == m, grow, big), axis=0, keepdims=True
        )
        idx = idxf.astype(jnp.int32)
        valid = m < _DIST_THRESH
        ids_ref[0, h, :] = (base + jnp.where(valid, idx, n))[0]
        mask_ref[0, h, :] = valid.astype(jnp.int32)[0]
        if h < _K - 1:
            dist = jnp.where(row == idx, jnp.float32(jnp.inf), dist)


def _topk_call(axy_t, det_boxes3d, interpret=False):
    B = axy_t.shape[0]
    T = axy_t.shape[2]
    N = det_boxes3d.shape[1]
    grid = (B, T // _TB)
    return pl.pallas_call(
        _topk_body,
        grid=grid,
        in_specs=[
            pl.BlockSpec((1, 2, _TB), lambda b, t: (b, 0, t)),
            pl.BlockSpec((1, N, 7), lambda b, t: (b, 0, 0)),
        ],
        out_specs=[
            pl.BlockSpec((1, _K, _TB), lambda b, t: (b, 0, t)),
            pl.BlockSpec((1, _K, _TB), lambda b, t: (b, 0, t)),
        ],
        out_shape=[
            jax.ShapeDtypeStruct((B, _K, T), jnp.int32),
            jax.ShapeDtypeStruct((B, _K, T), jnp.int32),
        ],
        interpret=interpret,
    )(axy_t, det_boxes3d)


def _make_sc_gather(num_rows, row_w, total):
    info = plsc.get_sparse_core_info()
    nc, ns = info.num_cores, info.num_subcores
    nw = nc * ns
    rpw = total // nw
    ch = rpw // 128
    mesh = plsc.VectorSubcoreMesh(core_axis_name="c", subcore_axis_name="s")

    @functools.partial(
        pl.kernel,
        mesh=mesh,
        out_type=jax.ShapeDtypeStruct((total, row_w), jnp.float32),
        scratch_types=[
            pltpu.VMEM((ch, 128), jnp.int32),
            pltpu.VMEM((rpw, row_w), jnp.float32),
            pltpu.SemaphoreType.DMA,
        ],
        compiler_params=pltpu.CompilerParams(use_tc_tiling_on_sc=False),
    )
    def gather_kernel(table_hbm, idx_hbm, out_hbm, idx_v, rows_v, sem):
        wid = lax.axis_index("s") * nc + lax.axis_index("c")
        pltpu.sync_copy(idx_hbm.at[wid], idx_v)
        copies = []
        for j in range(ch):
            copies.append(
                pltpu.async_copy(
                    table_hbm.at[idx_v.at[j]],
                    rows_v.at[pl.ds(j * 128, 128)],
                    sem,
                )
            )
        for c in copies:
            c.wait()
        pltpu.sync_copy(rows_v, out_hbm.at[pl.ds(wid * rpw, rpw)])

    return gather_kernel


def kernel(transfered_det, det_boxes3d, traj):
    B, T, _ = transfered_det.shape
    N = det_boxes3d.shape[1]
    L = traj.shape[1]

    axy_t = jnp.transpose(transfered_det[:, :, :2], (0, 2, 1))
    flat_ids, maskv = _topk_call(axy_t, det_boxes3d)

    table = jnp.pad(det_boxes3d, ((0, 0), (0, 1), (0, 9)))
    table = table.reshape(B * (N + 1), 16)

    nw = 32
    idx3 = flat_ids.reshape(nw, (B * T * _K) // (nw * 128), 128)
    gathered = _make_sc_gather(B * (N + 1), 16, B * T * _K)(table, idx3)
    boxes = jnp.transpose(
        gathered.reshape(B, _K, T, 16)[..., :8], (0, 2, 1, 3)
    )

    cand = jnp.concatenate([transfered_det[:, :, None, :], boxes], axis=2)
    global_candidates = cand[:, None]
    traj_rep = jnp.broadcast_to(
        traj[:, :, :, None, :], (B, L, T, _NUM_HYPO, traj.shape[-1])
    )
    hypotheses = jnp.concatenate([global_candidates, traj_rep], axis=1)
    valid_mask = jnp.transpose(maskv, (0, 2, 1)) != 0
    return (hypotheses, global_candidates, valid_mask)
